# Initial kernel scaffold; baseline (speedup 1.0000x reference)
import jax
import jax.numpy as jnp
from jax.experimental import pallas as pl


def kernel(x_nchw, w1, b1, w2, b2, w3, b3, wl1, bl1, wl2, bl2, wl3, bl3):
    raise NotImplementedError("write your pallas kernel here")


# speedup vs baseline: 1.1156x; 1.1156x over previous
import jax
import jax.numpy as jnp
from jax.experimental import pallas as pl
from jax.experimental.pallas import tpu as pltpu

H0, W0 = 64, 64
C1, C2, C3 = 32, 64, 128
FEAT = C3 * 8 * 8
HID = 256
NOUT = 2
TB = 8
MB = 256


def _conv_relu_pool(p_ref, h, w, w_ref, b_ref):
    tb, cin = p_ref.shape[0], p_ref.shape[-1]
    cout = w_ref.shape[-1]
    cols = [p_ref[:, dy:dy + h, dx:dx + w, :].reshape(tb * h * w, cin)
            for dy in range(3) for dx in range(3)]
    patch = jnp.concatenate(cols, axis=1)
    acc = jnp.dot(patch, w_ref[...], preferred_element_type=jnp.float32)
    y = jnp.maximum(acc + b_ref[...], 0.0)
    y = y.reshape(tb, h, w // 2, 2, cout).max(axis=3)
    return y.reshape(tb, h // 2, 2, w // 2, cout).max(axis=2)


def _tower_kernel(x_ref, a1_ref, bb1_ref, w2_ref, b2_ref, w3_ref, b3_ref,
                  f_ref, l_ref, p2_ref, p3_ref):
    tb = x_ref.shape[0]

    xb = x_ref[...].astype(jnp.bfloat16)
    l_ref[:, 0:1, 0:64] = jnp.zeros((tb, 1, 64), jnp.bfloat16)
    l_ref[:, 1:64, 0:64] = xb[:, 0:63, :]
    l_ref[:, :, 64:128] = xb
    l_ref[:, 0:63, 128:192] = xb[:, 1:64, :]
    l_ref[:, 63:64, 128:192] = jnp.zeros((tb, 1, 64), jnp.bfloat16)
    acc = jnp.dot(l_ref[...].reshape(tb * H0, 3 * W0), a1_ref[...],
                  preferred_element_type=jnp.float32)
    y1 = jnp.maximum(acc + bb1_ref[...], 0.0)
    y1 = y1.reshape(tb, H0, W0 // 2, 2, C1).max(axis=3)
    y1 = y1.reshape(tb, H0 // 2, 2, W0 // 2, C1).max(axis=2)

    p2_ref[:, 0:1, :, :] = jnp.zeros((tb, 1, 34, C1), jnp.bfloat16)
    p2_ref[:, 33:34, :, :] = jnp.zeros((tb, 1, 34, C1), jnp.bfloat16)
    p2_ref[:, :, 0:1, :] = jnp.zeros((tb, 34, 1, C1), jnp.bfloat16)
    p2_ref[:, :, 33:34, :] = jnp.zeros((tb, 34, 1, C1), jnp.bfloat16)
    p2_ref[:, 1:33, 1:33, :] = y1.astype(jnp.bfloat16)
    y2 = _conv_relu_pool(p2_ref, 32, 32, w2_ref, b2_ref)

    p3_ref[:, 0:1, :, :] = jnp.zeros((tb, 1, 18, C2), jnp.bfloat16)
    p3_ref[:, 17:18, :, :] = jnp.zeros((tb, 1, 18, C2), jnp.bfloat16)
    p3_ref[:, :, 0:1, :] = jnp.zeros((tb, 18, 1, C2), jnp.bfloat16)
    p3_ref[:, :, 17:18, :] = jnp.zeros((tb, 18, 1, C2), jnp.bfloat16)
    p3_ref[:, 1:17, 1:17, :] = y2.astype(jnp.bfloat16)
    y3 = _conv_relu_pool(p3_ref, 16, 16, w3_ref, b3_ref)

    f_ref[...] = y3.reshape(tb, FEAT).astype(jnp.bfloat16)


def _mlp_kernel(f_ref, wl1_ref, bl1_ref, wl2_ref, bl2_ref, wl3_ref, bl3_ref,
                o_ref):
    h = jnp.dot(f_ref[...], wl1_ref[...], preferred_element_type=jnp.float32)
    h = jnp.maximum(h + bl1_ref[...], 0.0).astype(jnp.bfloat16)
    h = jnp.dot(h, wl2_ref[...], preferred_element_type=jnp.float32)
    h = jnp.maximum(h + bl2_ref[...], 0.0).astype(jnp.bfloat16)
    o_ref[...] = (jnp.dot(h, wl3_ref[...], preferred_element_type=jnp.float32)
                  + bl3_ref[...])


def _build_banded_w1(w1):
    a = jnp.zeros((3, H0, H0, C1), jnp.float32)
    iw = jnp.arange(H0)
    for dy in range(3):
        for dx in range(3):
            lo = max(0, 1 - dx)
            hi = min(H0, H0 + 1 - dx)
            w = iw[lo:hi]
            a = a.at[dy, w + dx - 1, w, :].set(
                jnp.broadcast_to(w1[dy * 3 + dx], (hi - lo, C1)))
    return a.reshape(3 * H0, H0 * C1).astype(jnp.bfloat16)


def kernel(x_nchw, w1, b1, w2, b2, w3, b3, wl1, bl1, wl2, bl2, wl3, bl3):
    B = x_nchw.shape[0]
    x = x_nchw.reshape(B, H0, W0)
    Bp = ((B + MB - 1) // MB) * MB
    if Bp != B:
        x = jnp.concatenate([x, jnp.zeros((Bp - B, H0, W0), x.dtype)], axis=0)

    a1 = _build_banded_w1(w1)
    bb1 = jnp.tile(b1, W0).reshape(1, W0 * C1)

    const2 = lambda b: (0, 0)
    n_tiles = Bp // TB
    feat = pl.pallas_call(
        _tower_kernel,
        out_shape=jax.ShapeDtypeStruct((Bp, FEAT), jnp.bfloat16),
        grid=(n_tiles,),
        in_specs=[
            pl.BlockSpec((TB, H0, W0), lambda b: (b, 0, 0)),
            pl.BlockSpec((3 * H0, H0 * C1), const2),
            pl.BlockSpec((1, W0 * C1), const2),
            pl.BlockSpec((9 * C1, C2), const2), pl.BlockSpec((1, C2), const2),
            pl.BlockSpec((9 * C2, C3), const2), pl.BlockSpec((1, C3), const2),
        ],
        out_specs=pl.BlockSpec((TB, FEAT), lambda b: (b, 0)),
        scratch_shapes=[
            pltpu.VMEM((TB, H0, 3 * W0), jnp.bfloat16),
            pltpu.VMEM((TB, 34, 34, C1), jnp.bfloat16),
            pltpu.VMEM((TB, 18, 18, C2), jnp.bfloat16),
        ],
        compiler_params=pltpu.CompilerParams(
            dimension_semantics=("parallel",),
            vmem_limit_bytes=48 * 1024 * 1024,
        ),
    )(x, a1, bb1, w2, b2, w3, b3)

    out = pl.pallas_call(
        _mlp_kernel,
        out_shape=jax.ShapeDtypeStruct((Bp, NOUT), jnp.float32),
        grid=(Bp // MB,),
        in_specs=[
            pl.BlockSpec((MB, FEAT), lambda b: (b, 0)),
            pl.BlockSpec((FEAT, HID), const2), pl.BlockSpec((1, HID), const2),
            pl.BlockSpec((HID, HID), const2),  pl.BlockSpec((1, HID), const2),
            pl.BlockSpec((HID, NOUT), const2), pl.BlockSpec((1, NOUT), const2),
        ],
        out_specs=pl.BlockSpec((MB, NOUT), lambda b: (b, 0)),
        compiler_params=pltpu.CompilerParams(
            dimension_semantics=("parallel",),
            vmem_limit_bytes=48 * 1024 * 1024,
        ),
    )(feat, wl1, bl1, wl2, bl2, wl3, bl3)
    return out[:B]


# speedup vs baseline: 1.2062x; 1.0813x over previous
import jax
import jax.numpy as jnp
from jax.experimental import pallas as pl
from jax.experimental.pallas import tpu as pltpu

H0, W0 = 64, 64
C1, C2, C3 = 32, 64, 128
FEAT = C3 * 8 * 8
HID = 256
NOUT = 2
TB = 8
MB = 256


def _tower_kernel(x_ref, a1_ref, bb1_ref, a2_ref, bb2_ref, a3_ref, bb3_ref,
                  f_ref, l1_ref, l2_ref, l3_ref):
    tb = x_ref.shape[0]
    z = jnp.zeros

    xb = x_ref[...].astype(jnp.bfloat16)
    l1_ref[...] = z(l1_ref.shape, jnp.bfloat16)
    l1_ref[:, 1:64, 0:64] = xb[:, 0:63, :]
    l1_ref[:, :, 128:192] = xb
    l1_ref[:, 0:63, 256:320] = xb[:, 1:64, :]
    acc = jnp.dot(l1_ref[...].reshape(tb * 64, 384), a1_ref[...],
                  preferred_element_type=jnp.float32)
    y = jnp.maximum(acc + bb1_ref[...], 0.0)
    y = y.reshape(tb, 64, 32, 2, C1).max(axis=3)
    y = y.reshape(tb, 32, 2, 32, C1).max(axis=2)
    yb = y.reshape(tb, 32, 32 * C1).astype(jnp.bfloat16)

    l2_ref[:, 0:1, 0:1024] = z((tb, 1, 1024), jnp.bfloat16)
    l2_ref[:, 1:32, 0:1024] = yb[:, 0:31, :]
    l2_ref[:, :, 1024:2048] = yb
    l2_ref[:, 0:31, 2048:3072] = yb[:, 1:32, :]
    l2_ref[:, 31:32, 2048:3072] = z((tb, 1, 1024), jnp.bfloat16)
    acc2 = jnp.dot(l2_ref[...].reshape(tb * 32, 3072), a2_ref[...],
                   preferred_element_type=jnp.float32)
    y2 = jnp.maximum(acc2 + bb2_ref[...], 0.0)
    y2 = y2.reshape(tb, 32, 16, 2, C2).max(axis=3)
    y2 = y2.reshape(tb, 16, 2, 16, C2).max(axis=2)
    y2b = y2.reshape(tb, 16, 16 * C2).astype(jnp.bfloat16)

    l3_ref[:, 0:1, 0:1024] = z((tb, 1, 1024), jnp.bfloat16)
    l3_ref[:, 1:16, 0:1024] = y2b[:, 0:15, :]
    l3_ref[:, :, 1024:2048] = y2b
    l3_ref[:, 0:15, 2048:3072] = y2b[:, 1:16, :]
    l3_ref[:, 15:16, 2048:3072] = z((tb, 1, 1024), jnp.bfloat16)
    acc3 = jnp.dot(l3_ref[...].reshape(tb * 16, 3072), a3_ref[...],
                   preferred_element_type=jnp.float32)
    y3 = jnp.maximum(acc3 + bb3_ref[...], 0.0)
    y3 = y3.reshape(tb, 16, 8, 2, C3).max(axis=3)
    y3 = y3.reshape(tb, 8, 2, 8, C3).max(axis=2)
    f_ref[...] = y3.reshape(tb, FEAT).astype(jnp.bfloat16)


def _mlp_kernel(f_ref, wl1_ref, bl1_ref, wl2_ref, bl2_ref, wl3_ref, bl3_ref,
                o_ref):
    h = jnp.dot(f_ref[...], wl1_ref[...], preferred_element_type=jnp.float32)
    h = jnp.maximum(h + bl1_ref[...], 0.0).astype(jnp.bfloat16)
    h = jnp.dot(h, wl2_ref[...], preferred_element_type=jnp.float32)
    h = jnp.maximum(h + bl2_ref[...], 0.0).astype(jnp.bfloat16)
    o_ref[...] = (jnp.dot(h, wl3_ref[...], preferred_element_type=jnp.float32)
                  + bl3_ref[...])


def _banded_mat(wmat, n, cin, cout, chunk_pad=None):
    w9 = wmat.astype(jnp.float32).reshape(3, 3, cin, cout)
    iw = jnp.arange(n)
    masks = jnp.stack([(iw[:, None] == iw[None, :] + dx - 1)
                       .astype(jnp.float32) for dx in range(3)])
    a = jnp.einsum('xuw,dxio->duiwo', masks, w9)
    a = a.reshape(3, n * cin, n * cout)
    if chunk_pad is not None and chunk_pad > n * cin:
        a = jnp.pad(a, ((0, 0), (0, chunk_pad - n * cin), (0, 0)))
    return a.reshape(-1, n * cout).astype(jnp.bfloat16)


def kernel(x_nchw, w1, b1, w2, b2, w3, b3, wl1, bl1, wl2, bl2, wl3, bl3):
    B = x_nchw.shape[0]
    x = x_nchw.reshape(B, H0, W0)
    Bp = ((B + MB - 1) // MB) * MB
    if Bp != B:
        x = jnp.concatenate([x, jnp.zeros((Bp - B, H0, W0), x.dtype)], axis=0)

    a1 = _banded_mat(w1, 64, 1, C1, chunk_pad=128)
    a2 = _banded_mat(w2, 32, C1, C2)
    a3 = _banded_mat(w3, 16, C2, C3)
    bb1 = jnp.tile(b1.reshape(-1), 64).reshape(1, 64 * C1)
    bb2 = jnp.tile(b2.reshape(-1), 32).reshape(1, 32 * C2)
    bb3 = jnp.tile(b3.reshape(-1), 16).reshape(1, 16 * C3)

    const2 = lambda b: (0, 0)
    n_tiles = Bp // TB
    feat = pl.pallas_call(
        _tower_kernel,
        out_shape=jax.ShapeDtypeStruct((Bp, FEAT), jnp.bfloat16),
        grid=(n_tiles,),
        in_specs=[
            pl.BlockSpec((TB, H0, W0), lambda b: (b, 0, 0)),
            pl.BlockSpec((384, 64 * C1), const2),
            pl.BlockSpec((1, 64 * C1), const2),
            pl.BlockSpec((3072, 32 * C2), const2),
            pl.BlockSpec((1, 32 * C2), const2),
            pl.BlockSpec((3072, 16 * C3), const2),
            pl.BlockSpec((1, 16 * C3), const2),
        ],
        out_specs=pl.BlockSpec((TB, FEAT), lambda b: (b, 0)),
        scratch_shapes=[
            pltpu.VMEM((TB, 64, 384), jnp.bfloat16),
            pltpu.VMEM((TB, 32, 3072), jnp.bfloat16),
            pltpu.VMEM((TB, 16, 3072), jnp.bfloat16),
        ],
        compiler_params=pltpu.CompilerParams(
            dimension_semantics=("parallel",),
            vmem_limit_bytes=60 * 1024 * 1024,
        ),
    )(x, a1, bb1, a2, bb2, a3, bb3)

    out = pl.pallas_call(
        _mlp_kernel,
        out_shape=jax.ShapeDtypeStruct((Bp, NOUT), jnp.float32),
        grid=(Bp // MB,),
        in_specs=[
            pl.BlockSpec((MB, FEAT), lambda b: (b, 0)),
            pl.BlockSpec((FEAT, HID), const2), pl.BlockSpec((1, HID), const2),
            pl.BlockSpec((HID, HID), const2),  pl.BlockSpec((1, HID), const2),
            pl.BlockSpec((HID, NOUT), const2), pl.BlockSpec((1, NOUT), const2),
        ],
        out_specs=pl.BlockSpec((MB, NOUT), lambda b: (b, 0)),
        compiler_params=pltpu.CompilerParams(
            dimension_semantics=("parallel",),
            vmem_limit_bytes=48 * 1024 * 1024,
        ),
    )(feat, wl1, bl1, wl2, bl2, wl3, bl3)
    return out[:B]


# speedup vs baseline: 5.7587x; 4.7742x over previous
import jax
import jax.numpy as jnp
from jax.experimental import pallas as pl
from jax.experimental.pallas import tpu as pltpu

H0, W0 = 64, 64
C1, C2, C3 = 32, 64, 128
FEAT = C3 * 8 * 8
HID = 256
NOUT = 2
TB = 8
MB = 256


def _pool_stage(acc, bb_ref, tb, h, s_ref):
    half = acc.shape[-1] // 2
    wm = jnp.maximum(acc[:, :half], acc[:, half:])
    s_ref[...] = wm.reshape(tb, h // 2, 2, half)
    hm = jnp.maximum(s_ref[:, :, 0, :], s_ref[:, :, 1, :])
    return jnp.maximum(hm + bb_ref[...], 0.0).astype(jnp.bfloat16)


def _tower_kernel(x_ref, a1_ref, bb1_ref, a2_ref, bb2_ref, a3_ref, bb3_ref,
                  f_ref, l1_ref, l2_ref, l3_ref, s1_ref, s2_ref, s3_ref):
    tb = x_ref.shape[0]
    z = jnp.zeros

    xb = x_ref[...].astype(jnp.bfloat16)
    l1_ref[...] = z(l1_ref.shape, jnp.bfloat16)
    l1_ref[:, 1:64, 0:64] = xb[:, 0:63, :]
    l1_ref[:, :, 128:192] = xb
    l1_ref[:, 0:63, 256:320] = xb[:, 1:64, :]
    acc = jnp.dot(l1_ref[...].reshape(tb * 64, 384), a1_ref[...],
                  preferred_element_type=jnp.float32)
    yb = _pool_stage(acc, bb1_ref, tb, 64, s1_ref)

    l2_ref[:, 0:1, 0:1024] = z((tb, 1, 1024), jnp.bfloat16)
    l2_ref[:, 1:32, 0:1024] = yb[:, 0:31, :]
    l2_ref[:, :, 1024:2048] = yb
    l2_ref[:, 0:31, 2048:3072] = yb[:, 1:32, :]
    l2_ref[:, 31:32, 2048:3072] = z((tb, 1, 1024), jnp.bfloat16)
    acc2 = jnp.dot(l2_ref[...].reshape(tb * 32, 3072), a2_ref[...],
                   preferred_element_type=jnp.float32)
    y2b = _pool_stage(acc2, bb2_ref, tb, 32, s2_ref)

    l3_ref[:, 0:1, 0:1024] = z((tb, 1, 1024), jnp.bfloat16)
    l3_ref[:, 1:16, 0:1024] = y2b[:, 0:15, :]
    l3_ref[:, :, 1024:2048] = y2b
    l3_ref[:, 0:15, 2048:3072] = y2b[:, 1:16, :]
    l3_ref[:, 15:16, 2048:3072] = z((tb, 1, 1024), jnp.bfloat16)
    acc3 = jnp.dot(l3_ref[...].reshape(tb * 16, 3072), a3_ref[...],
                   preferred_element_type=jnp.float32)
    y3b = _pool_stage(acc3, bb3_ref, tb, 16, s3_ref)

    f_ref[...] = y3b.reshape(tb, FEAT)


def _mlp_kernel(f_ref, wl1_ref, bl1_ref, wl2_ref, bl2_ref, wl3_ref, bl3_ref,
                o_ref):
    h = jnp.dot(f_ref[...], wl1_ref[...], preferred_element_type=jnp.float32)
    h = jnp.maximum(h + bl1_ref[...], 0.0).astype(jnp.bfloat16)
    h = jnp.dot(h, wl2_ref[...], preferred_element_type=jnp.float32)
    h = jnp.maximum(h + bl2_ref[...], 0.0).astype(jnp.bfloat16)
    o_ref[...] = (jnp.dot(h, wl3_ref[...], preferred_element_type=jnp.float32)
                  + bl3_ref[...])


def _banded_mat(wmat, n, cin, cout, chunk_pad=None):
    w9 = wmat.astype(jnp.float32).reshape(3, 3, cin, cout)
    iw = jnp.arange(n)
    masks = jnp.stack([(iw[:, None] == iw[None, :] + dx - 1)
                       .astype(jnp.float32) for dx in range(3)])
    a = jnp.einsum('xuw,dxio->duiwo', masks, w9)
    a = a.reshape(3, n * cin, n // 2, 2, cout)
    a = jnp.transpose(a, (0, 1, 3, 2, 4)).reshape(3, n * cin, n * cout)
    if chunk_pad is not None and chunk_pad > n * cin:
        a = jnp.pad(a, ((0, 0), (0, chunk_pad - n * cin), (0, 0)))
    return a.reshape(-1, n * cout).astype(jnp.bfloat16)


def kernel(x_nchw, w1, b1, w2, b2, w3, b3, wl1, bl1, wl2, bl2, wl3, bl3):
    B = x_nchw.shape[0]
    x = x_nchw.reshape(B, H0, W0)
    Bp = ((B + MB - 1) // MB) * MB
    if Bp != B:
        x = jnp.concatenate([x, jnp.zeros((Bp - B, H0, W0), x.dtype)], axis=0)

    a1 = _banded_mat(w1, 64, 1, C1, chunk_pad=128)
    a2 = _banded_mat(w2, 32, C1, C2)
    a3 = _banded_mat(w3, 16, C2, C3)
    bb1 = jnp.tile(b1.reshape(-1), 32).reshape(1, 32 * C1)
    bb2 = jnp.tile(b2.reshape(-1), 16).reshape(1, 16 * C2)
    bb3 = jnp.tile(b3.reshape(-1), 8).reshape(1, 8 * C3)

    const2 = lambda b: (0, 0)
    n_tiles = Bp // TB
    feat = pl.pallas_call(
        _tower_kernel,
        out_shape=jax.ShapeDtypeStruct((Bp, FEAT), jnp.bfloat16),
        grid=(n_tiles,),
        in_specs=[
            pl.BlockSpec((TB, H0, W0), lambda b: (b, 0, 0)),
            pl.BlockSpec((384, 2048), const2),
            pl.BlockSpec((1, 32 * C1), const2),
            pl.BlockSpec((3072, 2048), const2),
            pl.BlockSpec((1, 16 * C2), const2),
            pl.BlockSpec((3072, 2048), const2),
            pl.BlockSpec((1, 8 * C3), const2),
        ],
        out_specs=pl.BlockSpec((TB, FEAT), lambda b: (b, 0)),
        scratch_shapes=[
            pltpu.VMEM((TB, 64, 384), jnp.bfloat16),
            pltpu.VMEM((TB, 32, 3072), jnp.bfloat16),
            pltpu.VMEM((TB, 16, 3072), jnp.bfloat16),
            pltpu.VMEM((TB, 32, 2, 1024), jnp.float32),
            pltpu.VMEM((TB, 16, 2, 1024), jnp.float32),
            pltpu.VMEM((TB, 8, 2, 1024), jnp.float32),
        ],
        compiler_params=pltpu.CompilerParams(
            dimension_semantics=("parallel",),
            vmem_limit_bytes=60 * 1024 * 1024,
        ),
    )(x, a1, bb1, a2, bb2, a3, bb3)

    out = pl.pallas_call(
        _mlp_kernel,
        out_shape=jax.ShapeDtypeStruct((Bp, NOUT), jnp.float32),
        grid=(Bp // MB,),
        in_specs=[
            pl.BlockSpec((MB, FEAT), lambda b: (b, 0)),
            pl.BlockSpec((FEAT, HID), const2), pl.BlockSpec((1, HID), const2),
            pl.BlockSpec((HID, HID), const2),  pl.BlockSpec((1, HID), const2),
            pl.BlockSpec((HID, NOUT), const2), pl.BlockSpec((1, NOUT), const2),
        ],
        out_specs=pl.BlockSpec((MB, NOUT), lambda b: (b, 0)),
        compiler_params=pltpu.CompilerParams(
            dimension_semantics=("parallel",),
            vmem_limit_bytes=48 * 1024 * 1024,
        ),
    )(feat, wl1, bl1, wl2, bl2, wl3, bl3)
    return out[:B]


# speedup vs baseline: 5.8313x; 1.0126x over previous
import jax
import jax.numpy as jnp
from jax.experimental import pallas as pl
from jax.experimental.pallas import tpu as pltpu

H0, W0 = 64, 64
C1, C2, C3 = 32, 64, 128
FEAT = C3 * 8 * 8
HID = 256
NOUT = 2
TB = 8
MB = 256


def _pool_stage(acc, bb_ref, tb, h, s_ref):
    half = acc.shape[-1] // 2
    wm = jnp.maximum(acc[:, :half], acc[:, half:])
    s_ref[...] = wm.reshape(tb, h // 2, 2, half)
    hm = jnp.maximum(s_ref[:, :, 0, :], s_ref[:, :, 1, :])
    return jnp.maximum(hm + bb_ref[...], 0.0).astype(jnp.bfloat16)


def _tower_kernel(x_ref, a1_ref, bb1_ref, a2_ref, bb2_ref, a3_ref, bb3_ref,
                  f_ref, l1_ref, l2_ref, l3_ref, s1_ref, s2_ref, s3_ref):
    tb = x_ref.shape[0]
    z = jnp.zeros

    xb = x_ref[...].astype(jnp.bfloat16)
    l1_ref[...] = z(l1_ref.shape, jnp.bfloat16)
    l1_ref[:, 1:64, 0:64] = xb[:, 0:63, :]
    l1_ref[:, :, 128:192] = xb
    l1_ref[:, 0:63, 256:320] = xb[:, 1:64, :]
    acc = jnp.dot(l1_ref[...].reshape(tb * 64, 384), a1_ref[...],
                  preferred_element_type=jnp.float32)
    yb = _pool_stage(acc, bb1_ref, tb, 64, s1_ref)

    l2_ref[:, 0:1, 0:1024] = z((tb, 1, 1024), jnp.bfloat16)
    l2_ref[:, 1:32, 0:1024] = yb[:, 0:31, :]
    l2_ref[:, :, 1024:2048] = yb
    l2_ref[:, 0:31, 2048:3072] = yb[:, 1:32, :]
    l2_ref[:, 31:32, 2048:3072] = z((tb, 1, 1024), jnp.bfloat16)
    acc2 = jnp.dot(l2_ref[...].reshape(tb * 32, 3072), a2_ref[...],
                   preferred_element_type=jnp.float32)
    y2b = _pool_stage(acc2, bb2_ref, tb, 32, s2_ref)

    l3_ref[:, 0:1, 0:1024] = z((tb, 1, 1024), jnp.bfloat16)
    l3_ref[:, 1:16, 0:1024] = y2b[:, 0:15, :]
    l3_ref[:, :, 1024:2048] = y2b
    l3_ref[:, 0:15, 2048:3072] = y2b[:, 1:16, :]
    l3_ref[:, 15:16, 2048:3072] = z((tb, 1, 1024), jnp.bfloat16)
    acc3 = jnp.dot(l3_ref[...].reshape(tb * 16, 3072), a3_ref[...],
                   preferred_element_type=jnp.float32)
    y3b = _pool_stage(acc3, bb3_ref, tb, 16, s3_ref)

    f_ref[...] = y3b.reshape(tb, FEAT)


def _mlp_kernel(f_ref, wl1_ref, bl1_ref, wl2_ref, bl2_ref, wl3_ref, bl3_ref,
                o_ref):
    h = jnp.dot(f_ref[...], wl1_ref[...], preferred_element_type=jnp.float32)
    h = jnp.maximum(h + bl1_ref[...], 0.0).astype(jnp.bfloat16)
    h = jnp.dot(h, wl2_ref[...], preferred_element_type=jnp.float32)
    h = jnp.maximum(h + bl2_ref[...], 0.0).astype(jnp.bfloat16)
    o_ref[...] = (jnp.dot(h, wl3_ref[...], preferred_element_type=jnp.float32)
                  + bl3_ref[...])


def _banded_mat(wmat, n, cin, cout, chunk_pad=None):
    w9 = wmat.astype(jnp.bfloat16).reshape(3, 3, cin, cout)
    iw = jnp.arange(n)
    wcol = jnp.concatenate([2 * jnp.arange(n // 2), 2 * jnp.arange(n // 2) + 1])
    masks = jnp.stack([(iw[:, None] == wcol[None, :] + dx - 1)
                       .astype(jnp.bfloat16) for dx in range(3)])
    a = jnp.einsum('xuw,dxio->duiwo', masks, w9)
    a = a.reshape(3, n * cin, n * cout)
    if chunk_pad is not None and chunk_pad > n * cin:
        a = jnp.pad(a, ((0, 0), (0, chunk_pad - n * cin), (0, 0)))
    return a.reshape(-1, n * cout).astype(jnp.bfloat16)


def kernel(x_nchw, w1, b1, w2, b2, w3, b3, wl1, bl1, wl2, bl2, wl3, bl3):
    B = x_nchw.shape[0]
    x = x_nchw.reshape(B, H0, W0)
    Bp = ((B + MB - 1) // MB) * MB
    if Bp != B:
        x = jnp.concatenate([x, jnp.zeros((Bp - B, H0, W0), x.dtype)], axis=0)

    a1 = _banded_mat(w1, 64, 1, C1, chunk_pad=128)
    a2 = _banded_mat(w2, 32, C1, C2)
    a3 = _banded_mat(w3, 16, C2, C3)
    bb1 = jnp.tile(b1.reshape(-1), 32).reshape(1, 32 * C1)
    bb2 = jnp.tile(b2.reshape(-1), 16).reshape(1, 16 * C2)
    bb3 = jnp.tile(b3.reshape(-1), 8).reshape(1, 8 * C3)

    const2 = lambda b: (0, 0)
    n_tiles = Bp // TB
    feat = pl.pallas_call(
        _tower_kernel,
        out_shape=jax.ShapeDtypeStruct((Bp, FEAT), jnp.bfloat16),
        grid=(n_tiles,),
        in_specs=[
            pl.BlockSpec((TB, H0, W0), lambda b: (b, 0, 0)),
            pl.BlockSpec((384, 2048), const2),
            pl.BlockSpec((1, 32 * C1), const2),
            pl.BlockSpec((3072, 2048), const2),
            pl.BlockSpec((1, 16 * C2), const2),
            pl.BlockSpec((3072, 2048), const2),
            pl.BlockSpec((1, 8 * C3), const2),
        ],
        out_specs=pl.BlockSpec((TB, FEAT), lambda b: (b, 0)),
        scratch_shapes=[
            pltpu.VMEM((TB, 64, 384), jnp.bfloat16),
            pltpu.VMEM((TB, 32, 3072), jnp.bfloat16),
            pltpu.VMEM((TB, 16, 3072), jnp.bfloat16),
            pltpu.VMEM((TB, 32, 2, 1024), jnp.float32),
            pltpu.VMEM((TB, 16, 2, 1024), jnp.float32),
            pltpu.VMEM((TB, 8, 2, 1024), jnp.float32),
        ],
        compiler_params=pltpu.CompilerParams(
            dimension_semantics=("parallel",),
            vmem_limit_bytes=60 * 1024 * 1024,
        ),
    )(x, a1, bb1, a2, bb2, a3, bb3)

    out = pl.pallas_call(
        _mlp_kernel,
        out_shape=jax.ShapeDtypeStruct((Bp, NOUT), jnp.float32),
        grid=(Bp // MB,),
        in_specs=[
            pl.BlockSpec((MB, FEAT), lambda b: (b, 0)),
            pl.BlockSpec((FEAT, HID), const2), pl.BlockSpec((1, HID), const2),
            pl.BlockSpec((HID, HID), const2),  pl.BlockSpec((1, HID), const2),
            pl.BlockSpec((HID, NOUT), const2), pl.BlockSpec((1, NOUT), const2),
        ],
        out_specs=pl.BlockSpec((MB, NOUT), lambda b: (b, 0)),
        compiler_params=pltpu.CompilerParams(
            dimension_semantics=("parallel",),
            vmem_limit_bytes=48 * 1024 * 1024,
        ),
    )(feat, wl1, bl1, wl2, bl2, wl3, bl3)
    return out[:B]


# speedup vs baseline: 8.1727x; 1.4015x over previous
import jax
import jax.numpy as jnp
from jax.experimental import pallas as pl
from jax.experimental.pallas import tpu as pltpu

H0, W0 = 64, 64
C1, C2, C3 = 32, 64, 128
FEAT = C3 * 8 * 8
HID = 256
NOUT = 2
TB = 8
MB = 256
F32 = jnp.float32


def _tower_kernel(x_ref, a1_ref, bb1_ref, a2_ref, bb2_ref, a3_ref, bb3_ref,
                  f_ref, xp_ref, y1p_ref, y2p_ref):
    tb = x_ref.shape[1]
    zrow = jnp.zeros((tb, 64), jnp.bfloat16)
    zlane = jnp.zeros((tb, 1024), jnp.bfloat16)

    def band3(p_ref, a_ref, rows, chunk):
        acc = jnp.dot(p_ref[0:rows * tb, :], a_ref[0:chunk, :],
                      preferred_element_type=F32)
        acc += jnp.dot(p_ref[tb:(rows + 1) * tb, :],
                       a_ref[chunk:2 * chunk, :], preferred_element_type=F32)
        acc += jnp.dot(p_ref[2 * tb:(rows + 2) * tb, :],
                       a_ref[2 * chunk:3 * chunk, :],
                       preferred_element_type=F32)
        return acc

    def pool_bias_relu(acc, bb_ref, half_rows):
        half = acc.shape[-1] // 2
        wm = jnp.maximum(acc[:, :half], acc[:, half:])
        v = wm.reshape(half_rows, 2 * tb, half)
        hm = jnp.maximum(v[:, :tb, :], v[:, tb:, :])
        hm = hm.reshape(half_rows * tb, half)
        return jnp.maximum(hm + bb_ref[...], 0.0).astype(jnp.bfloat16)

    xp_ref[0:tb, :] = zrow
    xp_ref[65 * tb:66 * tb, :] = zrow
    xp_ref[tb:65 * tb, :] = x_ref[...].astype(jnp.bfloat16).reshape(64 * tb, 64)
    acc1 = band3(xp_ref, a1_ref, 64, 64)
    y1 = pool_bias_relu(acc1, bb1_ref, 32)

    y1p_ref[0:tb, :] = zlane
    y1p_ref[33 * tb:34 * tb, :] = zlane
    y1p_ref[tb:33 * tb, :] = y1
    acc2 = band3(y1p_ref, a2_ref, 32, 1024)
    y2 = pool_bias_relu(acc2, bb2_ref, 16)

    y2p_ref[0:tb, :] = zlane
    y2p_ref[17 * tb:18 * tb, :] = zlane
    y2p_ref[tb:17 * tb, :] = y2
    acc3 = band3(y2p_ref, a3_ref, 16, 1024)
    y3 = pool_bias_relu(acc3, bb3_ref, 8)

    f_ref[...] = y3.reshape(8, tb, 1024)


def _mlp_kernel(f_ref, wl1_ref, bl1_ref, wl2_ref, bl2_ref, wl3_ref, bl3_ref,
                o_ref):
    h = jnp.dot(f_ref[0], wl1_ref[0:1024, :], preferred_element_type=F32)
    for j in range(1, 8):
        h += jnp.dot(f_ref[j], wl1_ref[j * 1024:(j + 1) * 1024, :],
                     preferred_element_type=F32)
    h = jnp.maximum(h + bl1_ref[...], 0.0).astype(jnp.bfloat16)
    h = jnp.dot(h, wl2_ref[...], preferred_element_type=F32)
    h = jnp.maximum(h + bl2_ref[...], 0.0).astype(jnp.bfloat16)
    o_ref[...] = (jnp.dot(h, wl3_ref[...], preferred_element_type=F32)
                  + bl3_ref[...])


def _banded_mat(wmat, n, cin, cout):
    w9 = wmat.astype(jnp.bfloat16).reshape(3, 3, cin, cout)
    iw = jnp.arange(n)
    wcol = jnp.concatenate([2 * jnp.arange(n // 2), 2 * jnp.arange(n // 2) + 1])
    masks = jnp.stack([(iw[:, None] == wcol[None, :] + dx - 1)
                       .astype(jnp.bfloat16) for dx in range(3)])
    a = jnp.einsum('xuw,dxio->duiwo', masks, w9)
    return a.reshape(3 * n * cin, n * cout)


def kernel(x_nchw, w1, b1, w2, b2, w3, b3, wl1, bl1, wl2, bl2, wl3, bl3):
    B = x_nchw.shape[0]
    x = x_nchw.reshape(B, H0, W0)
    Bp = ((B + MB - 1) // MB) * MB
    if Bp != B:
        x = jnp.concatenate([x, jnp.zeros((Bp - B, H0, W0), x.dtype)], axis=0)
    xt = jnp.transpose(x, (1, 0, 2))

    a1 = _banded_mat(w1, 64, 1, C1)
    a2 = _banded_mat(w2, 32, C1, C2)
    a3 = _banded_mat(w3, 16, C2, C3)
    bb1 = jnp.tile(b1.reshape(-1), 32).reshape(1, 32 * C1)
    bb2 = jnp.tile(b2.reshape(-1), 16).reshape(1, 16 * C2)
    bb3 = jnp.tile(b3.reshape(-1), 8).reshape(1, 8 * C3)

    const2 = lambda b: (0, 0)
    feat = pl.pallas_call(
        _tower_kernel,
        out_shape=jax.ShapeDtypeStruct((8, Bp, 1024), jnp.bfloat16),
        grid=(Bp // TB,),
        in_specs=[
            pl.BlockSpec((H0, TB, W0), lambda b: (0, b, 0)),
            pl.BlockSpec((192, 2048), const2),
            pl.BlockSpec((1, 32 * C1), const2),
            pl.BlockSpec((3072, 2048), const2),
            pl.BlockSpec((1, 16 * C2), const2),
            pl.BlockSpec((3072, 2048), const2),
            pl.BlockSpec((1, 8 * C3), const2),
        ],
        out_specs=pl.BlockSpec((8, TB, 1024), lambda b: (0, b, 0)),
        scratch_shapes=[
            pltpu.VMEM((66 * TB, 64), jnp.bfloat16),
            pltpu.VMEM((34 * TB, 1024), jnp.bfloat16),
            pltpu.VMEM((18 * TB, 1024), jnp.bfloat16),
        ],
        compiler_params=pltpu.CompilerParams(
            dimension_semantics=("parallel",),
            vmem_limit_bytes=60 * 1024 * 1024,
        ),
    )(xt, a1, bb1, a2, bb2, a3, bb3)

    out = pl.pallas_call(
        _mlp_kernel,
        out_shape=jax.ShapeDtypeStruct((Bp, NOUT), jnp.float32),
        grid=(Bp // MB,),
        in_specs=[
            pl.BlockSpec((8, MB, 1024), lambda b: (0, b, 0)),
            pl.BlockSpec((FEAT, HID), const2), pl.BlockSpec((1, HID), const2),
            pl.BlockSpec((HID, HID), const2),  pl.BlockSpec((1, HID), const2),
            pl.BlockSpec((HID, NOUT), const2), pl.BlockSpec((1, NOUT), const2),
        ],
        out_specs=pl.BlockSpec((MB, NOUT), lambda b: (b, 0)),
        compiler_params=pltpu.CompilerParams(
            dimension_semantics=("parallel",),
            vmem_limit_bytes=48 * 1024 * 1024,
        ),
    )(feat, wl1, bl1, wl2, bl2, wl3, bl3)
    return out[:B]
